# Initial kernel scaffold; baseline (speedup 1.0000x reference)
import jax
import jax.numpy as jnp
from jax.experimental import pallas as pl


def kernel(hidden_states, Wg, W1, W3, W2):
    raise NotImplementedError("write your pallas kernel here")


# speedup vs baseline: 1.3504x; 1.3504x over previous
import functools

import jax
import jax.numpy as jnp
from jax import lax
from jax.experimental import pallas as pl
from jax.experimental.pallas import tpu as pltpu

T = 2048
D = 1024
E = 8
K = 2
F = 2048
TM = 128
S = 4096 + E * TM
NT = S // TM


def _fiota(shape, dim):
    return lax.broadcasted_iota(jnp.int32, shape, dim).astype(jnp.float32)

def _router_body(x_ref, wg_ref, pos_ref, wsrt_ref, tsrc_ref, tmap_ref):
    x = x_ref[...]
    logits = jnp.dot(x, wg_ref[...], preferred_element_type=jnp.float32)
    m = jnp.max(logits, axis=1, keepdims=True)
    p = jnp.exp(logits - m)
    probs = p / jnp.sum(p, axis=1, keepdims=True)
    lane = _fiota( (T, E), 1)
    m1 = jnp.max(probs, axis=1, keepdims=True)
    i1 = jnp.min(jnp.where(probs == m1, lane, float(E)), axis=1, keepdims=True)
    masked = jnp.where(lane == i1, -1.0, probs)
    m2 = jnp.max(masked, axis=1, keepdims=True)
    i2 = jnp.min(jnp.where(masked == m2, lane, float(E)), axis=1, keepdims=True)

    oh1 = (lane == i1).astype(jnp.float32)
    oh2 = (lane == i2).astype(jnp.float32)

    counts = jnp.sum(oh1 + oh2, axis=0, keepdims=True)
    padded = jnp.ceil(counts * (1.0 / TM)) * TM
    r8 = _fiota( (E, E), 0)
    c8 = _fiota( (E, E), 1)
    su = (r8 < c8).astype(jnp.float32)
    offs = jnp.dot(padded, su, preferred_element_type=jnp.float32)
    ends = offs + padded

    rt = _fiota( (T, T), 0)
    ct = _fiota( (T, T), 1)
    ltri = (ct < rt).astype(jnp.float32)
    run = jnp.zeros((1, E), jnp.float32)
    pos_k = []
    for oh in (oh1, oh2):
        rk = jnp.dot(ltri, oh, preferred_element_type=jnp.float32) + run
        pos_k.append(jnp.sum(oh * (offs + rk), axis=1))
        run = run + jnp.sum(oh, axis=0, keepdims=True)
    pos0f, pos1f = pos_k
    pos_ref[0, :] = pos0f.astype(jnp.int32)
    pos_ref[1, :] = pos1f.astype(jnp.int32)
    for r in range(2, 8):
        pos_ref[r, :] = jnp.zeros((T,), jnp.int32)

    tvec = _fiota( (1, T), 1)
    w0r = jnp.reshape(m1, (1, T))
    w1r = jnp.reshape(m2, (1, T))
    for b in range(S // 512):
        sblk = _fiota( (1, 512), 1) + (512.0 * b)
        m0 = (jnp.reshape(pos0f, (T, 1)) == sblk).astype(jnp.float32)
        m1b = (jnp.reshape(pos1f, (T, 1)) == sblk).astype(jnp.float32)
        ts = jnp.dot(tvec, m0, preferred_element_type=jnp.float32) + \
             jnp.dot(tvec, m1b, preferred_element_type=jnp.float32)
        ws = jnp.dot(w0r, m0, preferred_element_type=jnp.float32) + \
             jnp.dot(w1r, m1b, preferred_element_type=jnp.float32)
        tsrc_ref[0, pl.ds(b * 512, 512)] = jnp.reshape(ts.astype(jnp.int32), (512,))
        wsrt_ref[0, pl.ds(b * 512, 512)] = jnp.reshape(ws, (512,))
    for r in range(1, 8):
        tsrc_ref[r, :] = jnp.zeros((S,), jnp.int32)
        wsrt_ref[r, :] = jnp.zeros((S,), jnp.float32)

    tl = _fiota( (1, 128), 1) * float(TM)
    acc = jnp.zeros((1, 128), jnp.float32)
    for e in range(E):
        acc = acc + (tl >= ends[:, e:e + 1]).astype(jnp.float32)
    tmap_ref[0, :] = jnp.minimum(acc, float(E - 1)).astype(jnp.int32)[0, :]
    for r in range(1, 8):
        tmap_ref[r, :] = jnp.zeros((128,), jnp.int32)


def _router(x2d, wg, interpret=False):
    return pl.pallas_call(
        _router_body,
        out_shape=(
            jax.ShapeDtypeStruct((8, T), jnp.int32),
            jax.ShapeDtypeStruct((8, S), jnp.float32),
            jax.ShapeDtypeStruct((8, S), jnp.int32),
            jax.ShapeDtypeStruct((8, 128), jnp.int32),
        ),
        interpret=interpret,
    )(x2d, wg)


def _gemm_body(tmap_ref, xs_ref, w1_ref, w3_ref, w2_ref, ws_ref, y_ref):
    xs = xs_ref[...]
    h = jnp.dot(xs, w1_ref[...], preferred_element_type=jnp.float32)
    g = jnp.dot(xs, w3_ref[...], preferred_element_type=jnp.float32)
    act = h * jax.nn.sigmoid(h) * g
    y = jnp.dot(act, w2_ref[...], preferred_element_type=jnp.float32)
    rr = _fiota( (TM, TM), 0)
    cc = _fiota( (TM, TM), 1)
    diag = jnp.where(rr == cc, ws_ref[...], 0.0)
    y_ref[...] = jnp.dot(diag, y, preferred_element_type=jnp.float32)


def _gemm(tmap, xs, w1, w3, w2, wsrt_rows, interpret=False):
    grid_spec = pltpu.PrefetchScalarGridSpec(
        num_scalar_prefetch=1,
        grid=(NT,),
        in_specs=[
            pl.BlockSpec((TM, D), lambda i, m: (i, 0)),
            pl.BlockSpec((None, D, F), lambda i, m: (m[i], 0, 0)),
            pl.BlockSpec((None, D, F), lambda i, m: (m[i], 0, 0)),
            pl.BlockSpec((None, F, D), lambda i, m: (m[i], 0, 0)),
            pl.BlockSpec((None, 1, TM), lambda i, m: (i, 0, 0)),
        ],
        out_specs=pl.BlockSpec((TM, D), lambda i, m: (i, 0)),
    )
    return pl.pallas_call(
        _gemm_body,
        grid_spec=grid_spec,
        out_shape=jax.ShapeDtypeStruct((S, D), jnp.float32),
        interpret=interpret,
    )(tmap, xs, w1, w3, w2, wsrt_rows)


def kernel(hidden_states, Wg, W1, W3, W2):
    orig_shape = hidden_states.shape
    x2d = hidden_states.reshape(-1, D)
    pos8, wsrt8, tsrc8, tmap8 = _router(x2d, Wg)
    tmap = tmap8[0, :NT]
    token_src = tsrc8[0]
    wsrt_rows = wsrt8[0].reshape(NT, 1, TM)

    xs = jnp.take(x2d, token_src, axis=0)

    y = _gemm(tmap, xs, W1, W3, W2, wsrt_rows)

    out = jnp.take(y, pos8[0], axis=0) + jnp.take(y, pos8[1], axis=0)
    return out.reshape(orig_shape)
